# baseline (speedup 1.0000x reference)
import functools

import jax
import jax.numpy as jnp
from jax.experimental import pallas as pl
from jax.experimental.pallas import tpu as pltpu

_B = 256
_T = 64
_DN = 2000
_PN = 1500
_AD = 32
_BB = 8


def _branch_kernel(lens_ref, tgt_ref, x_ref, w1_ref, b1_ref, w2_ref, b2_ref,
                   out_ref, *, d):
    x = x_ref[...]
    x2 = x.reshape(_BB * _T, d)
    h = jnp.tanh(
        jax.lax.dot_general(x2, w1_ref[...], (((1,), (0,)), ((), ())),
                            preferred_element_type=jnp.float32)
        + b1_ref[...])
    s = jax.lax.dot_general(h, w2_ref[...], (((1,), (0,)), ((), ())),
                            preferred_element_type=jnp.float32)
    s = s.reshape(_BB, _T) + b2_ref[0, 0]
    t_ids = jax.lax.broadcasted_iota(jnp.int32, (_BB, _T), 1)
    s = jnp.where(t_ids < lens_ref[...], s, -1e9)
    m = jnp.max(s, axis=1, keepdims=True)
    e = jnp.exp(s - m)
    p = e / jnp.sum(e, axis=1, keepdims=True)
    col = jax.lax.broadcasted_iota(jnp.int32, (_BB, _T, d), 2)
    hit = col == tgt_ref[...].reshape(_BB, 1, 1)
    out_ref[...] = jnp.minimum(x + jnp.where(hit, p[:, :, None], 0.0), 1.0)


def _row_spec():
    return pl.BlockSpec((_BB, 1), lambda i: (i, 0))


def _full_spec(shape):
    return pl.BlockSpec(shape, lambda i: tuple(0 for _ in shape))


def _run_branch(x, lens2, tgt2, w1, b1r, w2, b2r, d):
    return pl.pallas_call(
        functools.partial(_branch_kernel, d=d),
        grid=(_B // _BB,),
        compiler_params=pltpu.CompilerParams(
            dimension_semantics=("parallel",)),
        in_specs=[
            _row_spec(), _row_spec(),
            pl.BlockSpec((_BB, _T, d), lambda i: (i, 0, 0)),
            _full_spec((d, _AD)), _full_spec((1, _AD)),
            _full_spec((_AD, 1)), _full_spec((1, 1)),
        ],
        out_specs=pl.BlockSpec((_BB, _T, d), lambda i: (i, 0, 0)),
        out_shape=jax.ShapeDtypeStruct((_B, _T, d), jnp.float32),
    )(lens2, tgt2, x, w1, b1r, w2, b2r)


@jax.jit
def kernel(diagnosis_x, procedure_x, lens, target_diagnoses,
           target_procedures, Wd1, bd1, Wd2, bd2, Wp1, bp1, Wp2, bp2):
    lens2 = lens.astype(jnp.int32).reshape(_B, 1)
    tgtd2 = target_diagnoses.astype(jnp.int32).reshape(_B, 1)
    tgtp2 = target_procedures.astype(jnp.int32).reshape(_B, 1)
    dout = _run_branch(diagnosis_x, lens2, tgtd2, Wd1,
                       bd1.reshape(1, _AD), Wd2, bd2.reshape(1, 1), _DN)
    pout = _run_branch(procedure_x, lens2, tgtp2, Wp1,
                       bp1.reshape(1, _AD), Wp2, bp2.reshape(1, 1), _PN)
    return dout, pout
